# baseline (speedup 1.0000x reference)
import functools

import jax
import jax.numpy as jnp
from jax import lax
from jax.experimental import pallas as pl
from jax.experimental.pallas import tpu as pltpu
from jax.experimental.pallas import tpu_sc as plsc

N_LAYERS = 100000
CODE_DIM = 64
BATCH = 16384

NC = 2
NS = 16
NW = NC * NS
B_PER_W = BATCH // NW
CHUNK = 128
N_CHUNKS = B_PER_W // CHUNK

_mesh = plsc.VectorSubcoreMesh(core_axis_name="c", subcore_axis_name="s")


@functools.partial(
    pl.kernel,
    mesh=_mesh,
    out_type=jax.ShapeDtypeStruct((BATCH, CODE_DIM), jnp.float32),
    scratch_types=[
        pltpu.VMEM((B_PER_W,), jnp.int32),
        pltpu.VMEM((B_PER_W, CODE_DIM), jnp.float32),
        pltpu.SemaphoreType.DMA((N_CHUNKS,)),
        pltpu.SemaphoreType.DMA,
    ],
    compiler_params=pltpu.CompilerParams(skip_device_barrier=True),
)
def _gather_kernel(codes_hbm, idx_hbm, out_hbm, idx_v, rows_v, gsem, wsem):
    wid = lax.axis_index("s") * NC + lax.axis_index("c")
    base = wid * B_PER_W
    pltpu.sync_copy(idx_hbm.at[pl.ds(base, B_PER_W)], idx_v)

    for c in range(N_CHUNKS):

        def fire(g, _, c=c):
            j = c * CHUNK + g * 16
            v = idx_v[pl.ds(j, 16)]
            for k in range(16):
                pltpu.async_copy(codes_hbm.at[v[k]], rows_v.at[j + k], gsem.at[c])
            return _

        lax.fori_loop(0, CHUNK // 16, fire, 0)

    for c in range(N_CHUNKS):
        chunk = rows_v.at[pl.ds(c * CHUNK, CHUNK)]
        pltpu.make_async_copy(
            codes_hbm.at[pl.ds(0, CHUNK)], chunk, gsem.at[c]
        ).wait()
        pltpu.async_copy(chunk, out_hbm.at[pl.ds(base + c * CHUNK, CHUNK)], wsem)
    pltpu.make_async_copy(codes_hbm.at[pl.ds(0, B_PER_W)], rows_v, wsem).wait()


def kernel(layer_idx, codes):
    return _gather_kernel(codes, layer_idx)


# speedup vs baseline: 1.0132x; 1.0132x over previous
import functools

import jax
import jax.numpy as jnp
from jax import lax
from jax.experimental import pallas as pl
from jax.experimental.pallas import tpu as pltpu
from jax.experimental.pallas import tpu_sc as plsc

N_LAYERS = 100000
CODE_DIM = 64
BATCH = 16384

NC = 2
NS = 16
NW = NC * NS
B_PER_W = BATCH // NW

_mesh = plsc.VectorSubcoreMesh(core_axis_name="c", subcore_axis_name="s")


@functools.partial(
    pl.kernel,
    mesh=_mesh,
    out_type=jax.ShapeDtypeStruct((BATCH, CODE_DIM), jnp.float32),
    scratch_types=[
        pltpu.VMEM((B_PER_W,), jnp.int32),
        pltpu.VMEM((B_PER_W, CODE_DIM), jnp.float32),
        pltpu.SemaphoreType.DMA,
    ],
    compiler_params=pltpu.CompilerParams(use_tc_tiling_on_sc=True),
)
def _gather_kernel(codes_hbm, idx_hbm, out_hbm, idx_v, rows_v, sem):
    wid = lax.axis_index("s") * NC + lax.axis_index("c")
    base = wid * B_PER_W
    pltpu.sync_copy(idx_hbm.at[pl.ds(base, B_PER_W)], idx_v)

    def fire(g, _):
        v = idx_v[pl.ds(g * 16, 16)]
        for k in range(16):
            pltpu.async_copy(codes_hbm.at[v[k]], rows_v.at[g * 16 + k], sem)
        return _

    lax.fori_loop(0, B_PER_W // 16, fire, 0)
    pltpu.make_async_copy(codes_hbm.at[pl.ds(0, B_PER_W)], rows_v, sem).wait()
    pltpu.sync_copy(rows_v, out_hbm.at[pl.ds(base, B_PER_W)])


def kernel(layer_idx, codes):
    return _gather_kernel(codes, layer_idx)


# speedup vs baseline: 1.3798x; 1.3618x over previous
import functools

import jax
import jax.numpy as jnp
from jax import lax
from jax.experimental import pallas as pl
from jax.experimental.pallas import tpu as pltpu
from jax.experimental.pallas import tpu_sc as plsc

N_LAYERS = 100000
CODE_DIM = 64
BATCH = 16384

NC = 2
NS = 16
NW = NC * NS
N_PASSES = CODE_DIM // NW

_mesh = plsc.VectorSubcoreMesh(core_axis_name="c", subcore_axis_name="s")


@functools.partial(
    pl.kernel,
    mesh=_mesh,
    out_type=jax.ShapeDtypeStruct((CODE_DIM, BATCH), jnp.float32),
    scratch_types=[
        pltpu.VMEM((N_LAYERS,), jnp.float32),
        pltpu.VMEM((BATCH,), jnp.int32),
        pltpu.VMEM((BATCH // 2,), jnp.float32),
        pltpu.SemaphoreType.DMA,
    ],
    compiler_params=pltpu.CompilerParams(needs_layout_passes=False),
)
def _gather_kernel(codes_t_hbm, idx_hbm, out_t_hbm, row_v, idx_v, out_v, sem):
    wid = lax.axis_index("s") * NC + lax.axis_index("c")
    idx_cp = pltpu.async_copy(idx_hbm, idx_v, sem)
    row_cp = pltpu.async_copy(codes_t_hbm.at[wid], row_v, sem)
    idx_cp.wait()
    row_cp.wait()

    def gather_pass(f):
        for h in range(2):
            def gather(g, _, h=h):
                j = g * 16
                v = idx_v[pl.ds(h * (BATCH // 2) + j, 16)]
                out_v[pl.ds(j, 16)] = plsc.load_gather(row_v, [v])
                return _

            lax.fori_loop(0, BATCH // 32, gather, 0)
            pltpu.sync_copy(
                out_v, out_t_hbm.at[f, pl.ds(h * (BATCH // 2), BATCH // 2)]
            )

    gather_pass(wid)
    pltpu.sync_copy(codes_t_hbm.at[wid + NW], row_v)
    gather_pass(wid + NW)


def kernel(layer_idx, codes):
    return _gather_kernel(codes.T, layer_idx).T


# speedup vs baseline: 1.8062x; 1.3090x over previous
import functools

import jax
import jax.numpy as jnp
from jax import lax
from jax.experimental import pallas as pl
from jax.experimental.pallas import tpu as pltpu
from jax.experimental.pallas import tpu_sc as plsc

N_LAYERS = 100000
CODE_DIM = 64
BATCH = 16384

NC = 2
NS = 16
NW = NC * NS
N_PASSES = CODE_DIM // NW
OCHUNK = 4096
N_OCHUNKS = BATCH // OCHUNK

_mesh = plsc.VectorSubcoreMesh(core_axis_name="c", subcore_axis_name="s")


@functools.partial(
    pl.kernel,
    mesh=_mesh,
    out_type=jax.ShapeDtypeStruct((CODE_DIM, BATCH), jnp.float32),
    scratch_types=[
        pltpu.VMEM((N_LAYERS,), jnp.float32),
        pltpu.VMEM((BATCH,), jnp.int32),
        pltpu.VMEM((OCHUNK,), jnp.float32),
        pltpu.VMEM((OCHUNK,), jnp.float32),
        pltpu.SemaphoreType.DMA,
        pltpu.SemaphoreType.DMA,
    ],
    compiler_params=pltpu.CompilerParams(needs_layout_passes=False),
)
def _gather_kernel(codes_t_hbm, idx_hbm, out_t_hbm, row_v, idx_v, out_a, out_b,
                   rsem, wsem):
    wid = lax.axis_index("s") * NC + lax.axis_index("c")
    obufs = (out_a, out_b)
    drain_src = codes_t_hbm.at[0, pl.ds(0, OCHUNK)]

    idx_cp = pltpu.async_copy(idx_hbm, idx_v, rsem)
    row_cp = pltpu.async_copy(codes_t_hbm.at[wid], row_v, rsem)
    idx_cp.wait()
    row_cp.wait()

    for p in range(N_PASSES):
        f = wid + p * NW
        if p > 0:
            pltpu.sync_copy(codes_t_hbm.at[f], row_v)
        for q in range(N_OCHUNKS):
            k = p * N_OCHUNKS + q
            ob = obufs[k % 2]
            if k >= 2:
                pltpu.make_async_copy(drain_src, ob, wsem).wait()

            def gather(g, _, q=q, ob=ob):
                j = q * OCHUNK + g * 64
                vs = [idx_v[pl.ds(j + 16 * m, 16)] for m in range(4)]
                rs = [plsc.load_gather(row_v, [v]) for v in vs]
                for m in range(4):
                    ob[pl.ds(g * 64 + 16 * m, 16)] = rs[m]
                return _

            lax.fori_loop(0, OCHUNK // 64, gather, 0)
            pltpu.async_copy(
                ob, out_t_hbm.at[f, pl.ds(q * OCHUNK, OCHUNK)], wsem
            )
    pltpu.make_async_copy(drain_src, out_a, wsem).wait()
    pltpu.make_async_copy(drain_src, out_b, wsem).wait()


def kernel(layer_idx, codes):
    return _gather_kernel(codes.T, layer_idx).T
